# baseline (speedup 1.0000x reference)
import functools

import jax
import jax.numpy as jnp
from jax import lax
from jax.experimental import pallas as pl
from jax.experimental.pallas import tpu as pltpu
from jax.experimental.pallas import tpu_sc as plsc

_D = 64
_L = 200
_B = 4096
_LANES = 16
_NBUF = 4
_NTB = 2
_NC = _D // _LANES


@functools.cache
def _build_kernel():
    info = plsc.get_sparse_core_info()
    nc, ns = info.num_cores, info.num_subcores
    nw = nc * ns
    bb = _B // nw
    assert _B % nw == 0 and bb == 128

    mesh = plsc.VectorSubcoreMesh(core_axis_name="c", subcore_axis_name="s")

    @functools.partial(
        pl.kernel,
        out_type=jax.ShapeDtypeStruct((_L, _D // 8, nw, 8, 128), jnp.float32),
        mesh=mesh,
        scratch_types=[
            pltpu.VMEM((bb, _L), jnp.int32),
            pltpu.VMEM((_L, _D), jnp.float32),
            pltpu.VMEM((_NBUF, 128), jnp.int32),
            [pltpu.VMEM((bb, _D), jnp.float32) for _ in range(_NBUF)],
            [pltpu.VMEM((_D // 8, 1, 8, 128), jnp.float32)
             for _ in range(_NTB)],
            [pltpu.SemaphoreType.DMA for _ in range(_NBUF)],
            [pltpu.SemaphoreType.DMA for _ in range(_NTB)],
        ],
        compiler_params=pltpu.CompilerParams(use_tc_tiling_on_sc=False,
                                             needs_layout_passes=False),
    )
    def emb_kernel(ids_hbm, tok_hbm, pos_hbm, out_hbm, idx_p, pos_v, g_idx,
                   graw, t_v, gsems, osems):
        wid = lax.axis_index("s") * nc + lax.axis_index("c")
        base_b = wid * bb
        pltpu.sync_copy(ids_hbm.at[pl.ds(base_b, bb)], idx_p)
        pltpu.sync_copy(pos_hbm, pos_v)

        def build_gidx(b, l):
            iota = lax.iota(jnp.int32, _LANES)
            lsplat = jnp.full((_LANES,), l, jnp.int32)
            for j in range(bb // _LANES):
                v = plsc.load_gather(idx_p, [iota + (j * _LANES), lsplat])
                g_idx[b, pl.ds(j * _LANES, _LANES)] = v

        def start_gather(b):
            pltpu.async_copy(tok_hbm.at[g_idx.at[b]], graw[b], gsems[b])

        def wait_gather(b):
            pltpu.make_async_copy(tok_hbm.at[pl.ds(0, bb)], graw[b],
                                  gsems[b]).wait()

        def start_out(tb, l):
            pltpu.async_copy(t_v[tb], out_hbm.at[l, :, pl.ds(wid, 1)],
                             osems[tb])

        def wait_out(tb):
            pltpu.make_async_copy(t_v[tb], out_hbm.at[0, :, pl.ds(0, 1)],
                                  osems[tb]).wait()

        def transpose_add(b, tb, l):
            def body(k, carry):
                iota = lax.iota(jnp.int32, _LANES)
                zv = iota & 0
                dsub = iota & 7
                posv = [pos_v[l, pl.ds(c * _LANES, _LANES)]
                        for c in range(_NC)]
                dblk = [lax.shift_right_logical(iota + (c * _LANES), 3)
                        for c in range(_NC)]
                b0 = k * 8
                for bu in range(8):
                    bq = b0 + bu
                    bsplat = jnp.full((_LANES,), bq, jnp.int32)
                    for c in range(_NC):
                        v = graw[b][bq, pl.ds(c * _LANES, _LANES)] + posv[c]
                        plsc.store_scatter(
                            t_v[tb], [dblk[c], zv, dsub, bsplat], v)
                return carry
            lax.fori_loop(0, bb // 8, body, 0)

        def step(b, tb, l, first):
            wait_gather(b)
            if not first:
                wait_out(tb)
            transpose_add(b, tb, l)
            start_out(tb, l)
            lnext = jnp.minimum(l + _NBUF, _L - 1)
            build_gidx(b, lnext)
            start_gather(b)

        for l0 in range(_NBUF):
            build_gidx(l0, l0)
            start_gather(l0)
        for l0 in range(_NBUF):
            step(l0 % _NBUF, l0 % _NTB, l0, first=l0 < _NTB)

        def outer(i, carry):
            l0 = i * _NBUF
            for j in range(_NBUF):
                step(j, j % _NTB, l0 + j, first=False)
            return carry

        lax.fori_loop(1, _L // _NBUF, outer, 0)
        for b in range(_NBUF):
            wait_gather(b)
        for tb in range(_NTB):
            wait_out(tb)

    return emb_kernel


def kernel(input_ids, token_table, position_table):
    ids = input_ids.astype(jnp.int32)
    out5 = _build_kernel()(ids, token_table, position_table)
    return jnp.transpose(out5, (2, 4, 0, 1, 3)).reshape(_B, _L, _D)


# speedup vs baseline: 1.2673x; 1.2673x over previous
import functools

import jax
import jax.numpy as jnp
from jax import lax
from jax.experimental import pallas as pl
from jax.experimental.pallas import tpu as pltpu
from jax.experimental.pallas import tpu_sc as plsc

_D = 64
_L = 200
_B = 4096
_LANES = 16
_NBUF = 4
_NTB = 2
_NC = _D // _LANES


@functools.cache
def _build_kernel():
    info = plsc.get_sparse_core_info()
    nc, ns = info.num_cores, info.num_subcores
    nw = nc * ns
    bb = _B // nw
    assert _B % nw == 0 and bb == 128

    mesh = plsc.VectorSubcoreMesh(core_axis_name="c", subcore_axis_name="s")

    @functools.partial(
        pl.kernel,
        out_type=jax.ShapeDtypeStruct((_L, _D // 8, nw, 8, 128), jnp.float32),
        mesh=mesh,
        scratch_types=[
            pltpu.VMEM((bb, _L), jnp.int32),
            pltpu.VMEM((_L, _D), jnp.float32),
            pltpu.VMEM((_NBUF, 128), jnp.int32),
            [pltpu.VMEM((bb, _D), jnp.float32) for _ in range(_NBUF)],
            [pltpu.VMEM((_D // 8, 1, 8, 128), jnp.float32)
             for _ in range(_NTB)],
            [pltpu.SemaphoreType.DMA for _ in range(_NBUF)],
            [pltpu.SemaphoreType.DMA for _ in range(_NTB)],
        ],
        compiler_params=pltpu.CompilerParams(use_tc_tiling_on_sc=False,
                                             needs_layout_passes=False),
    )
    def emb_kernel(ids_hbm, tok_hbm, pos_hbm, out_hbm, idx_p, pos_v, g_idx,
                   graw, t_v, gsems, osems):
        wid = lax.axis_index("s") * nc + lax.axis_index("c")
        base_b = wid * bb
        pltpu.sync_copy(ids_hbm.at[pl.ds(base_b, bb)], idx_p)
        pltpu.sync_copy(pos_hbm, pos_v)

        def build_gidx(b, l):
            iota = lax.iota(jnp.int32, _LANES)
            lsplat = jnp.full((_LANES,), l, jnp.int32)
            for j in range(bb // _LANES):
                v = plsc.load_gather(idx_p, [iota + (j * _LANES), lsplat])
                g_idx[b, pl.ds(j * _LANES, _LANES)] = v

        def start_gather(b):
            pltpu.async_copy(tok_hbm.at[g_idx.at[b]], graw[b], gsems[b])

        def wait_gather(b):
            pltpu.make_async_copy(tok_hbm.at[pl.ds(0, bb)], graw[b],
                                  gsems[b]).wait()

        def start_out(tb, l):
            pltpu.async_copy(t_v[tb], out_hbm.at[l, :, pl.ds(wid, 1)],
                             osems[tb])

        def wait_out(tb):
            pltpu.make_async_copy(t_v[tb], out_hbm.at[0, :, pl.ds(0, 1)],
                                  osems[tb]).wait()

        def transpose_add(b, tb, l):
            @plsc.parallel_loop(0, bb, step=1, unroll=8)
            def body(bq):
                iota = lax.iota(jnp.int32, _LANES)
                zv = iota & 0
                dsub = iota & 7
                bsplat = jnp.full((_LANES,), bq, jnp.int32)
                for c in range(_NC):
                    posv = pos_v[l, pl.ds(c * _LANES, _LANES)]
                    dblk = lax.shift_right_logical(iota + (c * _LANES), 3)
                    v = graw[b][bq, pl.ds(c * _LANES, _LANES)] + posv
                    plsc.store_scatter(
                        t_v[tb], [dblk, zv, dsub, bsplat], v)

        def step(b, tb, l, first):
            wait_gather(b)
            if not first:
                wait_out(tb)
            transpose_add(b, tb, l)
            start_out(tb, l)
            lnext = jnp.minimum(l + _NBUF, _L - 1)
            build_gidx(b, lnext)
            start_gather(b)

        for l0 in range(_NBUF):
            build_gidx(l0, l0)
            start_gather(l0)
        for l0 in range(_NBUF):
            step(l0 % _NBUF, l0 % _NTB, l0, first=l0 < _NTB)

        def outer(i, carry):
            l0 = i * _NBUF
            for j in range(_NBUF):
                step(j, j % _NTB, l0 + j, first=False)
            return carry

        lax.fori_loop(1, _L // _NBUF, outer, 0)
        for b in range(_NBUF):
            wait_gather(b)
        for tb in range(_NTB):
            wait_out(tb)

    return emb_kernel


def kernel(input_ids, token_table, position_table):
    ids = input_ids.astype(jnp.int32)
    out5 = _build_kernel()(ids, token_table, position_table)
    return jnp.transpose(out5, (2, 4, 0, 1, 3)).reshape(_B, _L, _D)


# speedup vs baseline: 1.3139x; 1.0368x over previous
import functools

import jax
import jax.numpy as jnp
from jax import lax
from jax.experimental import pallas as pl
from jax.experimental.pallas import tpu as pltpu
from jax.experimental.pallas import tpu_sc as plsc

_D = 64
_L = 200
_B = 4096
_LANES = 16
_NBUF = 4
_NTB = 2
_NC = _D // _LANES


@functools.cache
def _build_kernel():
    info = plsc.get_sparse_core_info()
    nc, ns = info.num_cores, info.num_subcores
    nw = nc * ns
    bb = _B // nw
    assert _B % nw == 0 and bb == 128

    mesh = plsc.VectorSubcoreMesh(core_axis_name="c", subcore_axis_name="s")

    @functools.partial(
        pl.kernel,
        out_type=jax.ShapeDtypeStruct((_L * (_D // 8) * nw * 1024,),
                                      jnp.float32),
        mesh=mesh,
        scratch_types=[
            pltpu.VMEM((bb, _L), jnp.int32),
            pltpu.VMEM((_L, _D), jnp.float32),
            pltpu.VMEM((_NBUF, 128), jnp.int32),
            [pltpu.VMEM((bb, _D), jnp.float32) for _ in range(_NBUF)],
            [pltpu.VMEM((_D * 128,), jnp.float32) for _ in range(_NTB)],
            [pltpu.SemaphoreType.DMA for _ in range(_NBUF)],
            [pltpu.SemaphoreType.DMA for _ in range(_NTB)],
        ],
        compiler_params=pltpu.CompilerParams(use_tc_tiling_on_sc=False,
                                             needs_layout_passes=False),
    )
    def emb_kernel(ids_hbm, tok_hbm, pos_hbm, out_hbm, idx_p, pos_v, g_idx,
                   graw, t_v, gsems, osems):
        wid = lax.axis_index("s") * nc + lax.axis_index("c")
        base_b = wid * bb
        pltpu.sync_copy(ids_hbm.at[pl.ds(base_b, bb)], idx_p)
        pltpu.sync_copy(pos_hbm, pos_v)

        def build_gidx(b, l):
            iota = lax.iota(jnp.int32, _LANES)
            lsplat = jnp.full((_LANES,), l, jnp.int32)
            for j in range(bb // _LANES):
                v = plsc.load_gather(idx_p, [iota + (j * _LANES), lsplat])
                g_idx[b, pl.ds(j * _LANES, _LANES)] = v

        def start_gather(b):
            pltpu.async_copy(tok_hbm.at[g_idx.at[b]], graw[b], gsems[b])

        def wait_gather(b):
            pltpu.make_async_copy(tok_hbm.at[pl.ds(0, bb)], graw[b],
                                  gsems[b]).wait()

        def start_out(tb, l):
            for db in range(_D // 8):
                pltpu.async_copy(
                    t_v[tb].at[pl.ds(db * 1024, 1024)],
                    out_hbm.at[pl.ds((l * 256 + db * 32 + wid) * 1024, 1024)],
                    osems[tb])

        def wait_out(tb):
            pltpu.make_async_copy(t_v[tb], out_hbm.at[pl.ds(0, _D * 128)],
                                  osems[tb]).wait()

        def transpose_add(b, tb, l):
            iota = lax.iota(jnp.int32, _LANES)
            base_c = [(iota + c * _LANES) * 128 for c in range(_NC)]
            posv_c = [pos_v[l, pl.ds(c * _LANES, _LANES)] for c in range(_NC)]

            @plsc.parallel_loop(0, bb, step=1, unroll=8)
            def body(bq):
                for c in range(_NC):
                    v = graw[b][bq, pl.ds(c * _LANES, _LANES)] + posv_c[c]
                    plsc.store_scatter(t_v[tb], [base_c[c] + bq], v)

        def step(b, tb, l, first):
            wait_gather(b)
            if not first:
                wait_out(tb)
            transpose_add(b, tb, l)
            start_out(tb, l)
            lnext = jnp.minimum(l + _NBUF, _L - 1)
            build_gidx(b, lnext)
            start_gather(b)

        for l0 in range(_NBUF):
            build_gidx(l0, l0)
            start_gather(l0)
        for l0 in range(_NBUF):
            step(l0 % _NBUF, l0 % _NTB, l0, first=l0 < _NTB)

        def outer(i, carry):
            l0 = i * _NBUF
            for j in range(_NBUF):
                step(j, j % _NTB, l0 + j, first=False)
            return carry

        lax.fori_loop(1, _L // _NBUF, outer, 0)
        for b in range(_NBUF):
            wait_gather(b)
        for tb in range(_NTB):
            wait_out(tb)

    return emb_kernel


def kernel(input_ids, token_table, position_table):
    ids = input_ids.astype(jnp.int32)
    out = _build_kernel()(ids, token_table, position_table)
    out5 = out.reshape(_L, _D // 8, _B // 128, 8, 128)
    return jnp.transpose(out5, (2, 4, 0, 1, 3)).reshape(_B, _L, _D)


# speedup vs baseline: 2.1501x; 1.6364x over previous
import functools

import jax
import jax.numpy as jnp
from jax import lax
from jax.experimental import pallas as pl
from jax.experimental.pallas import tpu as pltpu
from jax.experimental.pallas import tpu_sc as plsc

_D = 64
_L = 200
_B = 4096
_LANES = 16
_NBUF = 4
_NTB = 2
_NC = _D // _LANES


@functools.cache
def _build_kernel():
    info = plsc.get_sparse_core_info()
    nc, ns = info.num_cores, info.num_subcores
    nw = nc * ns
    bb = _B // nw
    assert _B % nw == 0 and bb == 128

    mesh = plsc.VectorSubcoreMesh(core_axis_name="c", subcore_axis_name="s")

    @functools.partial(
        pl.kernel,
        out_type=jax.ShapeDtypeStruct((_L * (_D // 8) * nw, 8, 128),
                                      jnp.float32),
        mesh=mesh,
        scratch_types=[
            pltpu.VMEM((bb, _L), jnp.int32),
            pltpu.VMEM((_L, _D), jnp.float32),
            pltpu.VMEM((_NBUF, 128), jnp.int32),
            [pltpu.VMEM((bb, _D), jnp.float32) for _ in range(_NBUF)],
            [pltpu.VMEM((_D, 129), jnp.float32) for _ in range(_NTB)],
            [pltpu.SemaphoreType.DMA for _ in range(_NBUF)],
            [pltpu.SemaphoreType.DMA for _ in range(_NTB)],
        ],
        compiler_params=pltpu.CompilerParams(use_tc_tiling_on_sc=False,
                                             needs_layout_passes=False),
    )
    def emb_kernel(ids_hbm, tok_hbm, pos_hbm, out_hbm, idx_p, pos_v, g_idx,
                   graw, t_v, gsems, osems):
        wid = lax.axis_index("s") * nc + lax.axis_index("c")
        base_b = wid * bb
        pltpu.sync_copy(ids_hbm.at[pl.ds(base_b, bb)], idx_p)
        pltpu.sync_copy(pos_hbm, pos_v)

        def build_gidx(b, l):
            iota = lax.iota(jnp.int32, _LANES)
            lsplat = jnp.full((_LANES,), l, jnp.int32)
            for j in range(bb // _LANES):
                v = plsc.load_gather(idx_p, [iota + (j * _LANES), lsplat])
                g_idx[b, pl.ds(j * _LANES, _LANES)] = v

        def start_gather(b):
            pltpu.async_copy(tok_hbm.at[g_idx.at[b]], graw[b], gsems[b])

        def wait_gather(b):
            pltpu.make_async_copy(tok_hbm.at[pl.ds(0, bb)], graw[b],
                                  gsems[b]).wait()

        def start_out(tb, l):
            for db in range(_D // 8):
                pltpu.async_copy(
                    t_v[tb].at[pl.ds(db * 8, 8), pl.ds(0, 128)],
                    out_hbm.at[l * 256 + db * 32 + wid],
                    osems[tb])

        def wait_out(tb):
            for db in range(_D // 8):
                pltpu.make_async_copy(
                    t_v[tb].at[pl.ds(db * 8, 8), pl.ds(0, 128)],
                    out_hbm.at[0], osems[tb]).wait()

        def transpose_add(b, tb, l):
            iota = lax.iota(jnp.int32, _LANES)
            d_c = [iota + c * _LANES for c in range(_NC)]
            posv_c = [pos_v[l, pl.ds(c * _LANES, _LANES)] for c in range(_NC)]

            @plsc.parallel_loop(0, bb, step=1, unroll=8)
            def body(bq):
                bsplat = jnp.full((_LANES,), bq, jnp.int32)
                for c in range(_NC):
                    v = graw[b][bq, pl.ds(c * _LANES, _LANES)] + posv_c[c]
                    plsc.store_scatter(t_v[tb], [d_c[c], bsplat], v)

        def step(b, tb, l, first):
            wait_gather(b)
            if not first:
                wait_out(tb)
            transpose_add(b, tb, l)
            start_out(tb, l)
            lnext = jnp.minimum(l + _NBUF, _L - 1)
            build_gidx(b, lnext)
            start_gather(b)

        for l0 in range(_NBUF):
            build_gidx(l0, l0)
            start_gather(l0)
        for l0 in range(_NBUF):
            step(l0 % _NBUF, l0 % _NTB, l0, first=l0 < _NTB)

        def outer(i, carry):
            l0 = i * _NBUF
            for j in range(_NBUF):
                step(j, j % _NTB, l0 + j, first=False)
            return carry

        lax.fori_loop(1, _L // _NBUF, outer, 0)
        for b in range(_NBUF):
            wait_gather(b)
        for tb in range(_NTB):
            wait_out(tb)

    return emb_kernel


def kernel(input_ids, token_table, position_table):
    ids = input_ids.astype(jnp.int32)
    out = _build_kernel()(ids, token_table, position_table)
    out5 = out.reshape(_L, _D // 8, _B // 128, 8, 128)
    return jnp.transpose(out5, (2, 4, 0, 1, 3)).reshape(_B, _L, _D)
